# Initial kernel scaffold; baseline (speedup 1.0000x reference)
import functools

import jax
import jax.numpy as jnp
from jax import lax
from jax.experimental import pallas as pl
from jax.experimental.pallas import tpu as pltpu
from jax.experimental.pallas import tpu_sc as plsc

_N = 100000
_M = 100000
_T = 2000
_GRID = _M // _T
_DL = 16
_NW = 32
_CH = 3200
_NCH = 4
_BPW = _CH * _NCH
_BPAD = _NW * _BPW
_MPAD = _BPAD // 4
_EPS = 1e-5


def _sc_gather(table, idx):
  mesh = plsc.VectorSubcoreMesh(core_axis_name="c", subcore_axis_name="s")

  @functools.partial(
      pl.kernel,
      out_type=jax.ShapeDtypeStruct((_BPAD, _DL), jnp.float32),
      mesh=mesh,
      scratch_types=[
          pltpu.VMEM((_CH,), jnp.int32),
          pltpu.VMEM((_CH, _DL), jnp.float32),
          pltpu.SemaphoreType.DMA,
      ],
  )
  def k(table_hbm, idx_hbm, out_hbm, idx_v, rows_v, sem):
    wid = lax.axis_index("s") * 2 + lax.axis_index("c")
    base = wid * _BPW

    def step(i, carry):
      off = base + i * _CH
      pltpu.sync_copy(idx_hbm.at[pl.ds(off, _CH)], idx_v)
      pltpu.async_copy(table_hbm.at[idx_v], rows_v, sem).wait()
      pltpu.sync_copy(rows_v, out_hbm.at[pl.ds(off, _CH)])
      return carry

    lax.fori_loop(0, _NCH, step, 0)

  return k(table, idx)


def _norm_act(y, st_ref, g, be, kind, act, c_in):
  if kind == "inorm":
    mu = st_ref[0:1, :] / _M
    var = st_ref[1:2, :] / _M - mu * mu
    x = (y - mu) * lax.rsqrt(var + _EPS) * g + be
  elif kind == "gnorm":
    cnt = float(_M * c_in)
    mu = jnp.sum(st_ref[0:1, :]) / cnt
    var = jnp.sum(st_ref[1:2, :]) / cnt - mu * mu
    x = (y - mu) * lax.rsqrt(var + _EPS) * g + be
  else:
    x = y
  if act == "lrelu":
    x = jnp.where(x > 0, x, 0.01 * x)
  elif act == "relu":
    x = jnp.maximum(x, 0.0)
  return x


def _acc_stats(i, y, stout_ref):
  s0 = jnp.sum(y, axis=0, keepdims=True)
  s1 = jnp.sum(y * y, axis=0, keepdims=True)
  pad = jnp.zeros((6, y.shape[1]), jnp.float32)
  upd = jnp.concatenate([s0, s1, pad], axis=0)

  @pl.when(i == 0)
  def _():
    stout_ref[...] = jnp.zeros_like(stout_ref)

  stout_ref[...] += upd


def _row_spec(c):
  return pl.BlockSpec((_T, c), lambda i: (i, 0))


def _full_spec(r, c):
  return pl.BlockSpec((r, c), lambda i: (0, 0))


def _std_layer(y_in, stats, g, be, wt, b, kind, act, c_in, c_out):
  def body(y_ref, st_ref, g_ref, be_ref, w_ref, b_ref, out_ref, stout_ref):
    i = pl.program_id(0)
    x = _norm_act(y_ref[...], st_ref, g_ref[...], be_ref[...], kind, act, c_in)
    y = jnp.dot(x, w_ref[...], preferred_element_type=jnp.float32) + b_ref[...]
    out_ref[...] = y
    _acc_stats(i, y, stout_ref)

  return pl.pallas_call(
      body,
      grid=(_GRID,),
      in_specs=[_row_spec(c_in), _full_spec(8, c_in), _full_spec(1, c_in),
                _full_spec(1, c_in), _full_spec(c_in, c_out), _full_spec(1, c_out)],
      out_specs=[_row_spec(c_out), _full_spec(8, c_out)],
      out_shape=[jax.ShapeDtypeStruct((_M, c_out), jnp.float32),
                 jax.ShapeDtypeStruct((8, c_out), jnp.float32)],
      compiler_params=pltpu.CompilerParams(
          dimension_semantics=("arbitrary",)),
  )(y_in, stats, g, be, wt, b)


def kernel(features, faces, f_infos, params):
  p = params
  f32 = jnp.float32

  def wt(n):
    return p[n + "_W"].T

  def bias(n):
    return p[n + "_b"].reshape(1, -1)

  def gam(n):
    return p[n + "_g"].reshape(1, -1)

  def bet(n):
    return p[n + "_be"].reshape(1, -1)

  w_fe1 = p["fe1_W"]
  a_mat = jnp.zeros((4 * _DL, 128), f32)
  for v in range(4):
    a_mat = a_mat.at[v * _DL:v * _DL + 5, :].set(w_fe1[:, v * 5:(v + 1) * 5].T)
  bw_mat = jnp.zeros((_DL, 128), f32).at[:13, :].set(w_fe1[:, 20:33].T)

  head_w = p["head_W"]
  hw_blk = jnp.zeros((4 * _DL, 512), f32)
  for v in range(4):
    hw_blk = hw_blk.at[v * _DL:v * _DL + 5, v * 128:(v + 1) * 128].set(head_w)
  hb_blk = jnp.tile(p["head_b"], 4).reshape(1, 512)

  w_t1 = p["t1_W"]
  wt1a = w_t1[:, :256].T
  bt = w_t1[:, 256:].reshape(512, 128, 4).transpose(2, 1, 0).reshape(512, 512)

  table = jnp.pad(features, ((0, 0), (0, _DL - features.shape[1])))
  idx = jnp.pad(faces.reshape(-1).astype(jnp.int32), (0, _BPAD - 4 * _M))
  gz = _sc_gather(table, idx)
  g_faces = gz.reshape(_MPAD, 4 * _DL)
  f_pad = jnp.pad(f_infos, ((0, 0), (0, 3)))

  def k1_body(g_ref, f_ref, a_ref, bw_ref, b_ref, out_ref, stout_ref):
    i = pl.program_id(0)
    y = (jnp.dot(g_ref[...], a_ref[...], preferred_element_type=f32)
         + jnp.dot(f_ref[...], bw_ref[...], preferred_element_type=f32)
         + b_ref[...])
    out_ref[...] = y
    _acc_stats(i, y, stout_ref)

  y1, s1 = pl.pallas_call(
      k1_body,
      grid=(_GRID,),
      in_specs=[_row_spec(64), _row_spec(16), _full_spec(64, 128),
                _full_spec(16, 128), _full_spec(1, 128)],
      out_specs=[_row_spec(128), _full_spec(8, 128)],
      out_shape=[jax.ShapeDtypeStruct((_M, 128), f32),
                 jax.ShapeDtypeStruct((8, 128), f32)],
      compiler_params=pltpu.CompilerParams(dimension_semantics=("arbitrary",)),
  )(g_faces, f_pad, a_mat, bw_mat, bias("fe1"))

  y2, s2 = _std_layer(y1, s1, gam("fe1"), bet("fe1"), wt("fe2"), bias("fe2"),
                      "inorm", "lrelu", 128, 128)
  y3, s3 = _std_layer(y2, s2, gam("fe2"), bet("fe2"), wt("pr1"), bias("pr1"),
                      "inorm", "lrelu", 128, 256)
  y4, s4 = _std_layer(y3, s3, gam("pr1"), bet("pr1"), wt("pr2"), bias("pr2"),
                      "gnorm", "relu", 256, 512)
  y5, s5 = _std_layer(y4, s4, gam("pr2"), bet("pr2"), wt("pr3"), bias("pr3"),
                      "gnorm", "relu", 512, 512)
  y6, s6 = _std_layer(y5, s5, gam("pr3"), bet("pr3"), wt("pr4"), bias("pr4"),
                      "gnorm", "relu", 512, 256)

  def k7_body(y6_ref, s6_ref, y2_ref, s2_ref, g4_ref, be4_ref, w5_ref, b5_ref,
              g2_ref, be2_ref, w3_ref, b3_ref, out_ref, stout_ref):
    i = pl.program_id(0)
    x = _norm_act(y6_ref[...], s6_ref, g4_ref[...], be4_ref[...],
                  "gnorm", "relu", 256)
    r = jnp.dot(x, w5_ref[...], preferred_element_type=f32) + b5_ref[...]
    h = _norm_act(y2_ref[...], s2_ref, g2_ref[...], be2_ref[...],
                  "inorm", "lrelu", 128) + r
    y = jnp.dot(h, w3_ref[...], preferred_element_type=f32) + b3_ref[...]
    out_ref[...] = y
    _acc_stats(i, y, stout_ref)

  y7, s7 = pl.pallas_call(
      k7_body,
      grid=(_GRID,),
      in_specs=[_row_spec(256), _full_spec(8, 256), _row_spec(128),
                _full_spec(8, 128), _full_spec(1, 256), _full_spec(1, 256),
                _full_spec(256, 128), _full_spec(1, 128), _full_spec(1, 128),
                _full_spec(1, 128), _full_spec(128, 256), _full_spec(1, 256)],
      out_specs=[_row_spec(256), _full_spec(8, 256)],
      out_shape=[jax.ShapeDtypeStruct((_M, 256), f32),
                 jax.ShapeDtypeStruct((8, 256), f32)],
      compiler_params=pltpu.CompilerParams(dimension_semantics=("arbitrary",)),
  )(y6, s6, y2, s2, gam("pr4"), bet("pr4"), wt("pr5"), bias("pr5"),
    gam("fe2"), bet("fe2"), wt("fe3"), bias("fe3"))

  y8, s8 = _std_layer(y7, s7, gam("fe3"), bet("fe3"), wt("fe4"), bias("fe4"),
                      "inorm", "lrelu", 256, 512)

  def k9_body(y8_ref, s8_ref, g_ref, g4_ref, be4_ref, w5_ref, b5_ref,
              hw_ref, hb_ref, wa_ref, bt_ref, b1_ref, out_ref, stout_ref):
    i = pl.program_id(0)
    x = _norm_act(y8_ref[...], s8_ref, g4_ref[...], be4_ref[...],
                  "inorm", "lrelu", 512)
    xf1 = jnp.dot(x, w5_ref[...], preferred_element_type=f32) + b5_ref[...]
    gg = jnp.dot(g_ref[...], hw_ref[...], preferred_element_type=f32) + hb_ref[...]
    gg = jnp.where(gg > 0, gg, 0.1 * gg)
    y = (jnp.dot(xf1, wa_ref[...], preferred_element_type=f32)
         + jnp.dot(gg, bt_ref[...], preferred_element_type=f32) + b1_ref[...])
    out_ref[...] = y
    _acc_stats(i, y, stout_ref)

  y9, s9 = pl.pallas_call(
      k9_body,
      grid=(_GRID,),
      in_specs=[_row_spec(512), _full_spec(8, 512), _row_spec(64),
                _full_spec(1, 512), _full_spec(1, 512),
                _full_spec(512, 256), _full_spec(1, 256),
                _full_spec(64, 512), _full_spec(1, 512),
                _full_spec(256, 512), _full_spec(512, 512), _full_spec(1, 512)],
      out_specs=[_row_spec(512), _full_spec(8, 512)],
      out_shape=[jax.ShapeDtypeStruct((_M, 512), f32),
                 jax.ShapeDtypeStruct((8, 512), f32)],
      compiler_params=pltpu.CompilerParams(dimension_semantics=("arbitrary",)),
  )(y8, s8, g_faces, gam("fe4"), bet("fe4"), wt("fe5"), bias("fe5"),
    hw_blk, hb_blk, wt1a, bt, bias("t1"))

  y10, s10 = _std_layer(y9, s9, gam("t1"), bet("t1"), wt("t2"), bias("t2"),
                        "inorm", "relu", 512, 256)
  y11, s11 = _std_layer(y10, s10, gam("t2"), bet("t2"), wt("t3"), bias("t3"),
                        "inorm", "relu", 256, 128)
  y12, s12 = _std_layer(y11, s11, gam("t3"), bet("t3"), wt("t4"), bias("t4"),
                        "inorm", "relu", 128, 64)

  def k13_body(y_ref, st_ref, g_ref, be_ref, w_ref, b_ref, out_ref):
    x = _norm_act(y_ref[...], st_ref, g_ref[...], be_ref[...],
                  "inorm", "relu", 64)
    z = jnp.dot(x, w_ref[...], preferred_element_type=f32) + b_ref[...]
    zm = jnp.max(z, axis=1, keepdims=True)
    lse = zm + jnp.log(jnp.sum(jnp.exp(z - zm), axis=1, keepdims=True))
    out_ref[...] = z - lse

  out = pl.pallas_call(
      k13_body,
      grid=(_GRID,),
      in_specs=[_row_spec(64), _full_spec(8, 64), _full_spec(1, 64),
                _full_spec(1, 64), _full_spec(64, 2), _full_spec(1, 2)],
      out_specs=_row_spec(2),
      out_shape=jax.ShapeDtypeStruct((_M, 2), f32),
      compiler_params=pltpu.CompilerParams(dimension_semantics=("arbitrary",)),
  )(y12, s12, gam("t4"), bet("t4"), wt("t5"), bias("t5"))

  return out


# speedup vs baseline: 2.4004x; 2.4004x over previous
import functools

import jax
import jax.numpy as jnp
from jax import lax
from jax.experimental import pallas as pl
from jax.experimental.pallas import tpu as pltpu
from jax.experimental.pallas import tpu_sc as plsc

_N = 100000
_M = 100000
_T = 2000
_GRID = _M // _T
_DL = 16
_NW = 32
_CH = 3200
_NCH = 4
_BPW = _CH * _NCH
_BPAD = _NW * _BPW
_MPAD = _BPAD // 4
_EPS = 1e-5


def _sc_gather(table, idx):
  mesh = plsc.VectorSubcoreMesh(core_axis_name="c", subcore_axis_name="s")

  @functools.partial(
      pl.kernel,
      out_type=jax.ShapeDtypeStruct((_BPAD, _DL), jnp.float32),
      mesh=mesh,
      scratch_types=[
          pltpu.VMEM((_CH,), jnp.int32),
          pltpu.VMEM((_CH, _DL), jnp.float32),
          pltpu.SemaphoreType.DMA,
      ],
      compiler_params=pltpu.CompilerParams(use_tc_tiling_on_sc=False),
  )
  def k(table_hbm, idx_hbm, out_hbm, idx_v, rows_v, sem):
    wid = lax.axis_index("s") * 2 + lax.axis_index("c")
    base = wid * _BPW

    def step(i, carry):
      off = base + i * _CH
      pltpu.sync_copy(idx_hbm.at[pl.ds(off, _CH)], idx_v)
      pltpu.async_copy(table_hbm.at[idx_v], rows_v, sem).wait()
      pltpu.sync_copy(rows_v, out_hbm.at[pl.ds(off, _CH)])
      return carry

    lax.fori_loop(0, _NCH, step, 0)

  return k(table, idx)


def _norm_act(y, st_ref, g, be, kind, act, c_in):
  if kind == "inorm":
    mu = st_ref[0:1, :] / _M
    var = st_ref[1:2, :] / _M - mu * mu
    x = (y - mu) * lax.rsqrt(var + _EPS) * g + be
  elif kind == "gnorm":
    cnt = float(_M * c_in)
    mu = jnp.sum(st_ref[0:1, :]) / cnt
    var = jnp.sum(st_ref[1:2, :]) / cnt - mu * mu
    x = (y - mu) * lax.rsqrt(var + _EPS) * g + be
  else:
    x = y
  if act == "lrelu":
    x = jnp.where(x > 0, x, 0.01 * x)
  elif act == "relu":
    x = jnp.maximum(x, 0.0)
  return x


def _acc_stats(i, y, stout_ref):
  s0 = jnp.sum(y, axis=0, keepdims=True)
  s1 = jnp.sum(y * y, axis=0, keepdims=True)
  pad = jnp.zeros((6, y.shape[1]), jnp.float32)
  upd = jnp.concatenate([s0, s1, pad], axis=0)

  @pl.when(i == 0)
  def _():
    stout_ref[...] = jnp.zeros_like(stout_ref)

  stout_ref[...] += upd


def _row_spec(c):
  return pl.BlockSpec((_T, c), lambda i: (i, 0))


def _full_spec(r, c):
  return pl.BlockSpec((r, c), lambda i: (0, 0))


def _std_layer(y_in, stats, g, be, wt, b, kind, act, c_in, c_out):
  def body(y_ref, st_ref, g_ref, be_ref, w_ref, b_ref, out_ref, stout_ref):
    i = pl.program_id(0)
    x = _norm_act(y_ref[...], st_ref, g_ref[...], be_ref[...], kind, act, c_in)
    y = jnp.dot(x, w_ref[...], preferred_element_type=jnp.float32) + b_ref[...]
    out_ref[...] = y
    _acc_stats(i, y, stout_ref)

  return pl.pallas_call(
      body,
      grid=(_GRID,),
      in_specs=[_row_spec(c_in), _full_spec(8, c_in), _full_spec(1, c_in),
                _full_spec(1, c_in), _full_spec(c_in, c_out), _full_spec(1, c_out)],
      out_specs=[_row_spec(c_out), _full_spec(8, c_out)],
      out_shape=[jax.ShapeDtypeStruct((_M, c_out), jnp.float32),
                 jax.ShapeDtypeStruct((8, c_out), jnp.float32)],
      compiler_params=pltpu.CompilerParams(
          dimension_semantics=("arbitrary",)),
  )(y_in, stats, g, be, wt, b)


def kernel(features, faces, f_infos, params):
  p = params
  f32 = jnp.float32

  def wt(n):
    return p[n + "_W"].T

  def bias(n):
    return p[n + "_b"].reshape(1, -1)

  def gam(n):
    return p[n + "_g"].reshape(1, -1)

  def bet(n):
    return p[n + "_be"].reshape(1, -1)

  w_fe1 = p["fe1_W"]
  a_mat = jnp.zeros((4 * _DL, 128), f32)
  for v in range(4):
    a_mat = a_mat.at[v * _DL:v * _DL + 5, :].set(w_fe1[:, v * 5:(v + 1) * 5].T)
  bw_mat = jnp.zeros((_DL, 128), f32).at[:13, :].set(w_fe1[:, 20:33].T)

  head_w = p["head_W"]
  hw_blk = jnp.zeros((4 * _DL, 512), f32)
  for v in range(4):
    hw_blk = hw_blk.at[v * _DL:v * _DL + 5, v * 128:(v + 1) * 128].set(head_w)
  hb_blk = jnp.tile(p["head_b"], 4).reshape(1, 512)

  w_t1 = p["t1_W"]
  wt1a = w_t1[:, :256].T
  bt = w_t1[:, 256:].reshape(512, 128, 4).transpose(2, 1, 0).reshape(512, 512)

  table = jnp.pad(features, ((0, 0), (0, _DL - features.shape[1])))
  idx = jnp.pad(faces.reshape(-1).astype(jnp.int32), (0, _BPAD - 4 * _M))
  gz = _sc_gather(table, idx)
  g_faces = gz.reshape(_MPAD, 4 * _DL)
  f_pad = jnp.pad(f_infos, ((0, 0), (0, 3)))

  def k1_body(g_ref, f_ref, a_ref, bw_ref, b_ref, out_ref, stout_ref):
    i = pl.program_id(0)
    y = (jnp.dot(g_ref[...], a_ref[...], preferred_element_type=f32)
         + jnp.dot(f_ref[...], bw_ref[...], preferred_element_type=f32)
         + b_ref[...])
    out_ref[...] = y
    _acc_stats(i, y, stout_ref)

  y1, s1 = pl.pallas_call(
      k1_body,
      grid=(_GRID,),
      in_specs=[_row_spec(64), _row_spec(16), _full_spec(64, 128),
                _full_spec(16, 128), _full_spec(1, 128)],
      out_specs=[_row_spec(128), _full_spec(8, 128)],
      out_shape=[jax.ShapeDtypeStruct((_M, 128), f32),
                 jax.ShapeDtypeStruct((8, 128), f32)],
      compiler_params=pltpu.CompilerParams(dimension_semantics=("arbitrary",)),
  )(g_faces, f_pad, a_mat, bw_mat, bias("fe1"))

  y2, s2 = _std_layer(y1, s1, gam("fe1"), bet("fe1"), wt("fe2"), bias("fe2"),
                      "inorm", "lrelu", 128, 128)
  y3, s3 = _std_layer(y2, s2, gam("fe2"), bet("fe2"), wt("pr1"), bias("pr1"),
                      "inorm", "lrelu", 128, 256)
  y4, s4 = _std_layer(y3, s3, gam("pr1"), bet("pr1"), wt("pr2"), bias("pr2"),
                      "gnorm", "relu", 256, 512)
  y5, s5 = _std_layer(y4, s4, gam("pr2"), bet("pr2"), wt("pr3"), bias("pr3"),
                      "gnorm", "relu", 512, 512)
  y6, s6 = _std_layer(y5, s5, gam("pr3"), bet("pr3"), wt("pr4"), bias("pr4"),
                      "gnorm", "relu", 512, 256)

  def k7_body(y6_ref, s6_ref, y2_ref, s2_ref, g4_ref, be4_ref, w5_ref, b5_ref,
              g2_ref, be2_ref, w3_ref, b3_ref, out_ref, stout_ref):
    i = pl.program_id(0)
    x = _norm_act(y6_ref[...], s6_ref, g4_ref[...], be4_ref[...],
                  "gnorm", "relu", 256)
    r = jnp.dot(x, w5_ref[...], preferred_element_type=f32) + b5_ref[...]
    h = _norm_act(y2_ref[...], s2_ref, g2_ref[...], be2_ref[...],
                  "inorm", "lrelu", 128) + r
    y = jnp.dot(h, w3_ref[...], preferred_element_type=f32) + b3_ref[...]
    out_ref[...] = y
    _acc_stats(i, y, stout_ref)

  y7, s7 = pl.pallas_call(
      k7_body,
      grid=(_GRID,),
      in_specs=[_row_spec(256), _full_spec(8, 256), _row_spec(128),
                _full_spec(8, 128), _full_spec(1, 256), _full_spec(1, 256),
                _full_spec(256, 128), _full_spec(1, 128), _full_spec(1, 128),
                _full_spec(1, 128), _full_spec(128, 256), _full_spec(1, 256)],
      out_specs=[_row_spec(256), _full_spec(8, 256)],
      out_shape=[jax.ShapeDtypeStruct((_M, 256), f32),
                 jax.ShapeDtypeStruct((8, 256), f32)],
      compiler_params=pltpu.CompilerParams(dimension_semantics=("arbitrary",)),
  )(y6, s6, y2, s2, gam("pr4"), bet("pr4"), wt("pr5"), bias("pr5"),
    gam("fe2"), bet("fe2"), wt("fe3"), bias("fe3"))

  y8, s8 = _std_layer(y7, s7, gam("fe3"), bet("fe3"), wt("fe4"), bias("fe4"),
                      "inorm", "lrelu", 256, 512)

  def k9_body(y8_ref, s8_ref, g_ref, g4_ref, be4_ref, w5_ref, b5_ref,
              hw_ref, hb_ref, wa_ref, bt_ref, b1_ref, out_ref, stout_ref):
    i = pl.program_id(0)
    x = _norm_act(y8_ref[...], s8_ref, g4_ref[...], be4_ref[...],
                  "inorm", "lrelu", 512)
    xf1 = jnp.dot(x, w5_ref[...], preferred_element_type=f32) + b5_ref[...]
    gg = jnp.dot(g_ref[...], hw_ref[...], preferred_element_type=f32) + hb_ref[...]
    gg = jnp.where(gg > 0, gg, 0.1 * gg)
    y = (jnp.dot(xf1, wa_ref[...], preferred_element_type=f32)
         + jnp.dot(gg, bt_ref[...], preferred_element_type=f32) + b1_ref[...])
    out_ref[...] = y
    _acc_stats(i, y, stout_ref)

  y9, s9 = pl.pallas_call(
      k9_body,
      grid=(_GRID,),
      in_specs=[_row_spec(512), _full_spec(8, 512), _row_spec(64),
                _full_spec(1, 512), _full_spec(1, 512),
                _full_spec(512, 256), _full_spec(1, 256),
                _full_spec(64, 512), _full_spec(1, 512),
                _full_spec(256, 512), _full_spec(512, 512), _full_spec(1, 512)],
      out_specs=[_row_spec(512), _full_spec(8, 512)],
      out_shape=[jax.ShapeDtypeStruct((_M, 512), f32),
                 jax.ShapeDtypeStruct((8, 512), f32)],
      compiler_params=pltpu.CompilerParams(dimension_semantics=("arbitrary",)),
  )(y8, s8, g_faces, gam("fe4"), bet("fe4"), wt("fe5"), bias("fe5"),
    hw_blk, hb_blk, wt1a, bt, bias("t1"))

  y10, s10 = _std_layer(y9, s9, gam("t1"), bet("t1"), wt("t2"), bias("t2"),
                        "inorm", "relu", 512, 256)
  y11, s11 = _std_layer(y10, s10, gam("t2"), bet("t2"), wt("t3"), bias("t3"),
                        "inorm", "relu", 256, 128)
  y12, s12 = _std_layer(y11, s11, gam("t3"), bet("t3"), wt("t4"), bias("t4"),
                        "inorm", "relu", 128, 64)

  def k13_body(y_ref, st_ref, g_ref, be_ref, w_ref, b_ref, out_ref):
    x = _norm_act(y_ref[...], st_ref, g_ref[...], be_ref[...],
                  "inorm", "relu", 64)
    z = jnp.dot(x, w_ref[...], preferred_element_type=f32) + b_ref[...]
    zm = jnp.max(z, axis=1, keepdims=True)
    lse = zm + jnp.log(jnp.sum(jnp.exp(z - zm), axis=1, keepdims=True))
    out_ref[...] = z - lse

  out = pl.pallas_call(
      k13_body,
      grid=(_GRID,),
      in_specs=[_row_spec(64), _full_spec(8, 64), _full_spec(1, 64),
                _full_spec(1, 64), _full_spec(64, 2), _full_spec(1, 2)],
      out_specs=_row_spec(2),
      out_shape=jax.ShapeDtypeStruct((_M, 2), f32),
      compiler_params=pltpu.CompilerParams(dimension_semantics=("arbitrary",)),
  )(y12, s12, gam("t4"), bet("t4"), wt("t5"), bias("t5"))

  return out


# speedup vs baseline: 2.6857x; 1.1188x over previous
import functools

import jax
import jax.numpy as jnp
from jax import lax
from jax.experimental import pallas as pl
from jax.experimental.pallas import tpu as pltpu
from jax.experimental.pallas import tpu_sc as plsc

_N = 100000
_M = 100000
_T = 2000
_GRID = _M // _T
_DL = 16
_NW = 32
_CH = 3200
_NCH = 4
_BPW = _CH * _NCH
_BPAD = _NW * _BPW
_MPAD = _BPAD // 4
_EPS = 1e-5


def _sc_gather(table, idx):
  mesh = plsc.VectorSubcoreMesh(core_axis_name="c", subcore_axis_name="s")

  @functools.partial(
      pl.kernel,
      out_type=jax.ShapeDtypeStruct((_BPAD, _DL), jnp.float32),
      mesh=mesh,
      scratch_types=[
          pltpu.VMEM((_CH,), jnp.int32),
          pltpu.VMEM((_CH, _DL), jnp.float32),
          pltpu.SemaphoreType.DMA,
      ],
      compiler_params=pltpu.CompilerParams(use_tc_tiling_on_sc=False),
  )
  def k(table_hbm, idx_hbm, out_hbm, idx_v, rows_v, sem):
    wid = lax.axis_index("s") * 2 + lax.axis_index("c")
    base = wid * _BPW

    def step(i, carry):
      off = base + i * _CH
      pltpu.sync_copy(idx_hbm.at[pl.ds(off, _CH)], idx_v)
      pltpu.async_copy(table_hbm.at[idx_v], rows_v, sem).wait()
      pltpu.sync_copy(rows_v, out_hbm.at[pl.ds(off, _CH)])
      return carry

    lax.fori_loop(0, _NCH, step, 0)

  return k(table, idx)


def _norm_act(y, st_ref, g, be, kind, act, c_in):
  y = y.astype(jnp.float32)
  if kind == "inorm":
    mu = st_ref[0:1, :] / _M
    var = st_ref[1:2, :] / _M - mu * mu
    x = (y - mu) * lax.rsqrt(var + _EPS) * g + be
  elif kind == "gnorm":
    cnt = float(_M * c_in)
    mu = jnp.sum(st_ref[0:1, :]) / cnt
    var = jnp.sum(st_ref[1:2, :]) / cnt - mu * mu
    x = (y - mu) * lax.rsqrt(var + _EPS) * g + be
  else:
    x = y
  if act == "lrelu":
    x = jnp.where(x > 0, x, 0.01 * x)
  elif act == "relu":
    x = jnp.maximum(x, 0.0)
  return x


def _acc_stats(i, y, stout_ref):
  y = y.astype(jnp.float32)
  s0 = jnp.sum(y, axis=0, keepdims=True)
  s1 = jnp.sum(y * y, axis=0, keepdims=True)
  pad = jnp.zeros((6, y.shape[1]), jnp.float32)
  upd = jnp.concatenate([s0, s1, pad], axis=0)

  @pl.when(i == 0)
  def _():
    stout_ref[...] = jnp.zeros_like(stout_ref)

  stout_ref[...] += upd


def _row_spec(c):
  return pl.BlockSpec((_T, c), lambda i: (i, 0))


def _full_spec(r, c):
  return pl.BlockSpec((r, c), lambda i: (0, 0))


def _std_layer(y_in, stats, g, be, wt, b, kind, act, c_in, c_out):
  def body(y_ref, st_ref, g_ref, be_ref, w_ref, b_ref, out_ref, stout_ref):
    i = pl.program_id(0)
    x = _norm_act(y_ref[...], st_ref, g_ref[...], be_ref[...], kind, act, c_in)
    y = jnp.dot(x.astype(jnp.bfloat16), w_ref[...],
                preferred_element_type=jnp.float32) + b_ref[...]
    yb = y.astype(jnp.bfloat16)
    out_ref[...] = yb
    _acc_stats(i, yb, stout_ref)

  return pl.pallas_call(
      body,
      grid=(_GRID,),
      in_specs=[_row_spec(c_in), _full_spec(8, c_in), _full_spec(1, c_in),
                _full_spec(1, c_in), _full_spec(c_in, c_out), _full_spec(1, c_out)],
      out_specs=[_row_spec(c_out), _full_spec(8, c_out)],
      out_shape=[jax.ShapeDtypeStruct((_M, c_out), jnp.bfloat16),
                 jax.ShapeDtypeStruct((8, c_out), jnp.float32)],
      compiler_params=pltpu.CompilerParams(
          dimension_semantics=("arbitrary",)),
  )(y_in, stats, g, be, wt, b)


def kernel(features, faces, f_infos, params):
  p = params
  f32 = jnp.float32

  def wt(n):
    return p[n + "_W"].T.astype(jnp.bfloat16)

  def bias(n):
    return p[n + "_b"].reshape(1, -1)

  def gam(n):
    return p[n + "_g"].reshape(1, -1)

  def bet(n):
    return p[n + "_be"].reshape(1, -1)

  w_fe1 = p["fe1_W"]
  a_mat = jnp.zeros((4 * _DL, 128), f32)
  for v in range(4):
    a_mat = a_mat.at[v * _DL:v * _DL + 5, :].set(w_fe1[:, v * 5:(v + 1) * 5].T)
  bw_mat = jnp.zeros((_DL, 128), f32).at[:13, :].set(w_fe1[:, 20:33].T)

  head_w = p["head_W"]
  hw_blk = jnp.zeros((4 * _DL, 512), f32)
  for v in range(4):
    hw_blk = hw_blk.at[v * _DL:v * _DL + 5, v * 128:(v + 1) * 128].set(head_w)
  hb_blk = jnp.tile(p["head_b"], 4).reshape(1, 512)

  w_t1 = p["t1_W"]
  wt1a = w_t1[:, :256].T.astype(jnp.bfloat16)
  bt = (w_t1[:, 256:].reshape(512, 128, 4).transpose(2, 1, 0)
        .reshape(512, 512).astype(jnp.bfloat16))

  table = jnp.pad(features, ((0, 0), (0, _DL - features.shape[1])))
  idx = jnp.pad(faces.reshape(-1).astype(jnp.int32), (0, _BPAD - 4 * _M))
  gz = _sc_gather(table, idx)
  g_faces = gz.reshape(_MPAD, 4 * _DL)
  f_pad = jnp.pad(f_infos, ((0, 0), (0, 3)))

  def k1_body(g_ref, f_ref, a_ref, bw_ref, b_ref, out_ref, stout_ref):
    i = pl.program_id(0)
    y = (jnp.dot(g_ref[...], a_ref[...], preferred_element_type=f32)
         + jnp.dot(f_ref[...], bw_ref[...], preferred_element_type=f32)
         + b_ref[...])
    yb = y.astype(jnp.bfloat16)
    out_ref[...] = yb
    _acc_stats(i, yb, stout_ref)

  y1, s1 = pl.pallas_call(
      k1_body,
      grid=(_GRID,),
      in_specs=[_row_spec(64), _row_spec(16), _full_spec(64, 128),
                _full_spec(16, 128), _full_spec(1, 128)],
      out_specs=[_row_spec(128), _full_spec(8, 128)],
      out_shape=[jax.ShapeDtypeStruct((_M, 128), jnp.bfloat16),
                 jax.ShapeDtypeStruct((8, 128), f32)],
      compiler_params=pltpu.CompilerParams(dimension_semantics=("arbitrary",)),
  )(g_faces, f_pad, a_mat, bw_mat, bias("fe1"))

  y2, s2 = _std_layer(y1, s1, gam("fe1"), bet("fe1"), wt("fe2"), bias("fe2"),
                      "inorm", "lrelu", 128, 128)
  y3, s3 = _std_layer(y2, s2, gam("fe2"), bet("fe2"), wt("pr1"), bias("pr1"),
                      "inorm", "lrelu", 128, 256)
  y4, s4 = _std_layer(y3, s3, gam("pr1"), bet("pr1"), wt("pr2"), bias("pr2"),
                      "gnorm", "relu", 256, 512)
  y5, s5 = _std_layer(y4, s4, gam("pr2"), bet("pr2"), wt("pr3"), bias("pr3"),
                      "gnorm", "relu", 512, 512)
  y6, s6 = _std_layer(y5, s5, gam("pr3"), bet("pr3"), wt("pr4"), bias("pr4"),
                      "gnorm", "relu", 512, 256)

  def k7_body(y6_ref, s6_ref, y2_ref, s2_ref, g4_ref, be4_ref, w5_ref, b5_ref,
              g2_ref, be2_ref, w3_ref, b3_ref, out_ref, stout_ref):
    i = pl.program_id(0)
    x = _norm_act(y6_ref[...], s6_ref, g4_ref[...], be4_ref[...],
                  "gnorm", "relu", 256)
    r = jnp.dot(x.astype(jnp.bfloat16), w5_ref[...],
                preferred_element_type=f32) + b5_ref[...]
    h = _norm_act(y2_ref[...], s2_ref, g2_ref[...], be2_ref[...],
                  "inorm", "lrelu", 128) + r
    y = jnp.dot(h.astype(jnp.bfloat16), w3_ref[...],
                preferred_element_type=f32) + b3_ref[...]
    yb = y.astype(jnp.bfloat16)
    out_ref[...] = yb
    _acc_stats(i, yb, stout_ref)

  y7, s7 = pl.pallas_call(
      k7_body,
      grid=(_GRID,),
      in_specs=[_row_spec(256), _full_spec(8, 256), _row_spec(128),
                _full_spec(8, 128), _full_spec(1, 256), _full_spec(1, 256),
                _full_spec(256, 128), _full_spec(1, 128), _full_spec(1, 128),
                _full_spec(1, 128), _full_spec(128, 256), _full_spec(1, 256)],
      out_specs=[_row_spec(256), _full_spec(8, 256)],
      out_shape=[jax.ShapeDtypeStruct((_M, 256), jnp.bfloat16),
                 jax.ShapeDtypeStruct((8, 256), f32)],
      compiler_params=pltpu.CompilerParams(dimension_semantics=("arbitrary",)),
  )(y6, s6, y2, s2, gam("pr4"), bet("pr4"), wt("pr5"), bias("pr5"),
    gam("fe2"), bet("fe2"), wt("fe3"), bias("fe3"))

  y8, s8 = _std_layer(y7, s7, gam("fe3"), bet("fe3"), wt("fe4"), bias("fe4"),
                      "inorm", "lrelu", 256, 512)

  def k9_body(y8_ref, s8_ref, g_ref, g4_ref, be4_ref, w5_ref, b5_ref,
              hw_ref, hb_ref, wa_ref, bt_ref, b1_ref, out_ref, stout_ref):
    i = pl.program_id(0)
    x = _norm_act(y8_ref[...], s8_ref, g4_ref[...], be4_ref[...],
                  "inorm", "lrelu", 512)
    xf1 = jnp.dot(x.astype(jnp.bfloat16), w5_ref[...],
                  preferred_element_type=f32) + b5_ref[...]
    gg = jnp.dot(g_ref[...], hw_ref[...], preferred_element_type=f32) + hb_ref[...]
    gg = jnp.where(gg > 0, gg, 0.1 * gg)
    y = (jnp.dot(xf1.astype(jnp.bfloat16), wa_ref[...],
                 preferred_element_type=f32)
         + jnp.dot(gg.astype(jnp.bfloat16), bt_ref[...],
                   preferred_element_type=f32) + b1_ref[...])
    yb = y.astype(jnp.bfloat16)
    out_ref[...] = yb
    _acc_stats(i, yb, stout_ref)

  y9, s9 = pl.pallas_call(
      k9_body,
      grid=(_GRID,),
      in_specs=[_row_spec(512), _full_spec(8, 512), _row_spec(64),
                _full_spec(1, 512), _full_spec(1, 512),
                _full_spec(512, 256), _full_spec(1, 256),
                _full_spec(64, 512), _full_spec(1, 512),
                _full_spec(256, 512), _full_spec(512, 512), _full_spec(1, 512)],
      out_specs=[_row_spec(512), _full_spec(8, 512)],
      out_shape=[jax.ShapeDtypeStruct((_M, 512), jnp.bfloat16),
                 jax.ShapeDtypeStruct((8, 512), f32)],
      compiler_params=pltpu.CompilerParams(dimension_semantics=("arbitrary",)),
  )(y8, s8, g_faces, gam("fe4"), bet("fe4"), wt("fe5"), bias("fe5"),
    hw_blk, hb_blk, wt1a, bt, bias("t1"))

  y10, s10 = _std_layer(y9, s9, gam("t1"), bet("t1"), wt("t2"), bias("t2"),
                        "inorm", "relu", 512, 256)
  y11, s11 = _std_layer(y10, s10, gam("t2"), bet("t2"), wt("t3"), bias("t3"),
                        "inorm", "relu", 256, 128)
  y12, s12 = _std_layer(y11, s11, gam("t3"), bet("t3"), wt("t4"), bias("t4"),
                        "inorm", "relu", 128, 64)

  def k13_body(y_ref, st_ref, g_ref, be_ref, w_ref, b_ref, out_ref):
    x = _norm_act(y_ref[...], st_ref, g_ref[...], be_ref[...],
                  "inorm", "relu", 64)
    z = jnp.dot(x.astype(jnp.bfloat16), w_ref[...],
                preferred_element_type=f32) + b_ref[...]
    zm = jnp.max(z, axis=1, keepdims=True)
    lse = zm + jnp.log(jnp.sum(jnp.exp(z - zm), axis=1, keepdims=True))
    out_ref[...] = z - lse

  out = pl.pallas_call(
      k13_body,
      grid=(_GRID,),
      in_specs=[_row_spec(64), _full_spec(8, 64), _full_spec(1, 64),
                _full_spec(1, 64), _full_spec(64, 2), _full_spec(1, 2)],
      out_specs=_row_spec(2),
      out_shape=jax.ShapeDtypeStruct((_M, 2), f32),
      compiler_params=pltpu.CompilerParams(dimension_semantics=("arbitrary",)),
  )(y12, s12, gam("t4"), bet("t4"), wt("t5"), bias("t5"))

  return out


# speedup vs baseline: 3.1049x; 1.1561x over previous
import functools

import jax
import jax.numpy as jnp
from jax import lax
from jax.experimental import pallas as pl
from jax.experimental.pallas import tpu as pltpu
from jax.experimental.pallas import tpu_sc as plsc

_N = 100000
_M = 100000
_T = 5000
_GRID = _M // _T
_DL = 16
_NW = 32
_CH = 3200
_NCH = 4
_BPW = _CH * _NCH
_BPAD = _NW * _BPW
_MPAD = _BPAD // 4
_EPS = 1e-5


def _sc_gather(table, idx):
  mesh = plsc.VectorSubcoreMesh(core_axis_name="c", subcore_axis_name="s")

  @functools.partial(
      pl.kernel,
      out_type=jax.ShapeDtypeStruct((_BPAD, _DL), jnp.float32),
      mesh=mesh,
      scratch_types=[
          pltpu.VMEM((2, _CH), jnp.int32),
          pltpu.VMEM((2, _CH, _DL), jnp.float32),
          pltpu.SemaphoreType.DMA((2,)),
          pltpu.SemaphoreType.DMA((2,)),
      ],
      compiler_params=pltpu.CompilerParams(use_tc_tiling_on_sc=False),
  )
  def k(table_hbm, idx_hbm, out_hbm, idx_v, rows_v, gsem, ssem):
    wid = lax.axis_index("s") * 2 + lax.axis_index("c")
    base = wid * _BPW

    def start(i):
      b = i % 2
      pltpu.sync_copy(idx_hbm.at[pl.ds(base + i * _CH, _CH)], idx_v.at[b])
      return pltpu.async_copy(table_hbm.at[idx_v.at[b]], rows_v.at[b],
                              gsem.at[b])

    gath = [start(0), None]
    stor = [None, None]
    for i in range(_NCH):
      b, nb = i % 2, (i + 1) % 2
      if i + 1 < _NCH:
        if stor[nb] is not None:
          stor[nb].wait()
        gath[nb] = start(i + 1)
      gath[b].wait()
      stor[b] = pltpu.async_copy(rows_v.at[b],
                                 out_hbm.at[pl.ds(base + i * _CH, _CH)],
                                 ssem.at[b])
    stor[0].wait()
    stor[1].wait()

  return k(table, idx)


def _norm_act(y, st_ref, g, be, kind, act, c_in):
  y = y.astype(jnp.float32)
  if kind == "inorm":
    mu = st_ref[0:1, :] / _M
    var = st_ref[1:2, :] / _M - mu * mu
    x = (y - mu) * lax.rsqrt(var + _EPS) * g + be
  elif kind == "gnorm":
    cnt = float(_M * c_in)
    mu = jnp.sum(st_ref[0:1, :]) / cnt
    var = jnp.sum(st_ref[1:2, :]) / cnt - mu * mu
    x = (y - mu) * lax.rsqrt(var + _EPS) * g + be
  else:
    x = y
  if act == "lrelu":
    x = jnp.maximum(x, 0.01 * x)
  elif act == "relu":
    x = jnp.maximum(x, 0.0)
  return x


def _acc_stats(i, y, stout_ref):
  y = y.astype(jnp.float32)
  s0 = jnp.sum(y, axis=0, keepdims=True)
  s1 = jnp.sum(y * y, axis=0, keepdims=True)
  pad = jnp.zeros((6, y.shape[1]), jnp.float32)
  upd = jnp.concatenate([s0, s1, pad], axis=0)

  @pl.when(i == 0)
  def _():
    stout_ref[...] = jnp.zeros_like(stout_ref)

  stout_ref[...] += upd


def _row_spec(c):
  return pl.BlockSpec((_T, c), lambda i: (i, 0))


def _full_spec(r, c):
  return pl.BlockSpec((r, c), lambda i: (0, 0))


def _std_layer(y_in, stats, g, be, wt, b, kind, act, c_in, c_out):
  def body(y_ref, st_ref, g_ref, be_ref, w_ref, b_ref, out_ref, stout_ref):
    i = pl.program_id(0)
    x = _norm_act(y_ref[...], st_ref, g_ref[...], be_ref[...], kind, act, c_in)
    y = jnp.dot(x.astype(jnp.bfloat16), w_ref[...],
                preferred_element_type=jnp.float32) + b_ref[...]
    yb = y.astype(jnp.bfloat16)
    out_ref[...] = yb
    _acc_stats(i, yb, stout_ref)

  return pl.pallas_call(
      body,
      grid=(_GRID,),
      in_specs=[_row_spec(c_in), _full_spec(8, c_in), _full_spec(1, c_in),
                _full_spec(1, c_in), _full_spec(c_in, c_out), _full_spec(1, c_out)],
      out_specs=[_row_spec(c_out), _full_spec(8, c_out)],
      out_shape=[jax.ShapeDtypeStruct((_M, c_out), jnp.bfloat16),
                 jax.ShapeDtypeStruct((8, c_out), jnp.float32)],
      compiler_params=pltpu.CompilerParams(
          dimension_semantics=("arbitrary",)),
  )(y_in, stats, g, be, wt, b)


def kernel(features, faces, f_infos, params):
  p = params
  f32 = jnp.float32

  def wt(n):
    return p[n + "_W"].T.astype(jnp.bfloat16)

  def bias(n):
    return p[n + "_b"].reshape(1, -1)

  def gam(n):
    return p[n + "_g"].reshape(1, -1)

  def bet(n):
    return p[n + "_be"].reshape(1, -1)

  w_fe1 = p["fe1_W"]
  a_mat = jnp.zeros((4 * _DL, 128), f32)
  for v in range(4):
    a_mat = a_mat.at[v * _DL:v * _DL + 5, :].set(w_fe1[:, v * 5:(v + 1) * 5].T)
  a_mat = a_mat.astype(jnp.bfloat16)
  bw_mat = (jnp.zeros((_DL, 128), f32).at[:13, :].set(w_fe1[:, 20:33].T)
            .astype(jnp.bfloat16))

  head_w = p["head_W"]
  hw_blk = jnp.zeros((4 * _DL, 512), f32)
  for v in range(4):
    hw_blk = hw_blk.at[v * _DL:v * _DL + 5, v * 128:(v + 1) * 128].set(head_w)
  hw_blk = hw_blk.astype(jnp.bfloat16)
  hb_blk = jnp.tile(p["head_b"], 4).reshape(1, 512)

  w_t1 = p["t1_W"]
  wt1a = w_t1[:, :256].T.astype(jnp.bfloat16)
  bt = (w_t1[:, 256:].reshape(512, 128, 4).transpose(2, 1, 0)
        .reshape(512, 512).astype(jnp.bfloat16))

  table = jnp.pad(features, ((0, 0), (0, _DL - features.shape[1])))
  idx = jnp.pad(faces.reshape(-1).astype(jnp.int32), (0, _BPAD - 4 * _M))
  gz = _sc_gather(table, idx)
  g_faces = gz.reshape(_MPAD, 4 * _DL)
  f_pad = jnp.pad(f_infos, ((0, 0), (0, 3)))

  def k1_body(g_ref, f_ref, a_ref, bw_ref, b_ref, out_ref, stout_ref):
    i = pl.program_id(0)
    y = (jnp.dot(g_ref[...].astype(jnp.bfloat16), a_ref[...],
                 preferred_element_type=f32)
         + jnp.dot(f_ref[...].astype(jnp.bfloat16), bw_ref[...],
                   preferred_element_type=f32)
         + b_ref[...])
    yb = y.astype(jnp.bfloat16)
    out_ref[...] = yb
    _acc_stats(i, yb, stout_ref)

  y1, s1 = pl.pallas_call(
      k1_body,
      grid=(_GRID,),
      in_specs=[_row_spec(64), _row_spec(16), _full_spec(64, 128),
                _full_spec(16, 128), _full_spec(1, 128)],
      out_specs=[_row_spec(128), _full_spec(8, 128)],
      out_shape=[jax.ShapeDtypeStruct((_M, 128), jnp.bfloat16),
                 jax.ShapeDtypeStruct((8, 128), f32)],
      compiler_params=pltpu.CompilerParams(dimension_semantics=("arbitrary",)),
  )(g_faces, f_pad, a_mat, bw_mat, bias("fe1"))

  y2, s2 = _std_layer(y1, s1, gam("fe1"), bet("fe1"), wt("fe2"), bias("fe2"),
                      "inorm", "lrelu", 128, 128)
  y3, s3 = _std_layer(y2, s2, gam("fe2"), bet("fe2"), wt("pr1"), bias("pr1"),
                      "inorm", "lrelu", 128, 256)
  y4, s4 = _std_layer(y3, s3, gam("pr1"), bet("pr1"), wt("pr2"), bias("pr2"),
                      "gnorm", "relu", 256, 512)
  y5, s5 = _std_layer(y4, s4, gam("pr2"), bet("pr2"), wt("pr3"), bias("pr3"),
                      "gnorm", "relu", 512, 512)
  y6, s6 = _std_layer(y5, s5, gam("pr3"), bet("pr3"), wt("pr4"), bias("pr4"),
                      "gnorm", "relu", 512, 256)

  def k7_body(y6_ref, s6_ref, y2_ref, s2_ref, g4_ref, be4_ref, w5_ref, b5_ref,
              g2_ref, be2_ref, w3_ref, b3_ref, out_ref, stout_ref):
    i = pl.program_id(0)
    x = _norm_act(y6_ref[...], s6_ref, g4_ref[...], be4_ref[...],
                  "gnorm", "relu", 256)
    r = jnp.dot(x.astype(jnp.bfloat16), w5_ref[...],
                preferred_element_type=f32) + b5_ref[...]
    h = _norm_act(y2_ref[...], s2_ref, g2_ref[...], be2_ref[...],
                  "inorm", "lrelu", 128) + r
    y = jnp.dot(h.astype(jnp.bfloat16), w3_ref[...],
                preferred_element_type=f32) + b3_ref[...]
    yb = y.astype(jnp.bfloat16)
    out_ref[...] = yb
    _acc_stats(i, yb, stout_ref)

  y7, s7 = pl.pallas_call(
      k7_body,
      grid=(_GRID,),
      in_specs=[_row_spec(256), _full_spec(8, 256), _row_spec(128),
                _full_spec(8, 128), _full_spec(1, 256), _full_spec(1, 256),
                _full_spec(256, 128), _full_spec(1, 128), _full_spec(1, 128),
                _full_spec(1, 128), _full_spec(128, 256), _full_spec(1, 256)],
      out_specs=[_row_spec(256), _full_spec(8, 256)],
      out_shape=[jax.ShapeDtypeStruct((_M, 256), jnp.bfloat16),
                 jax.ShapeDtypeStruct((8, 256), f32)],
      compiler_params=pltpu.CompilerParams(dimension_semantics=("arbitrary",)),
  )(y6, s6, y2, s2, gam("pr4"), bet("pr4"), wt("pr5"), bias("pr5"),
    gam("fe2"), bet("fe2"), wt("fe3"), bias("fe3"))

  y8, s8 = _std_layer(y7, s7, gam("fe3"), bet("fe3"), wt("fe4"), bias("fe4"),
                      "inorm", "lrelu", 256, 512)

  def k9_body(y8_ref, s8_ref, g_ref, g4_ref, be4_ref, w5_ref, b5_ref,
              hw_ref, hb_ref, wa_ref, bt_ref, b1_ref, out_ref, stout_ref):
    i = pl.program_id(0)
    x = _norm_act(y8_ref[...], s8_ref, g4_ref[...], be4_ref[...],
                  "inorm", "lrelu", 512)
    xf1 = jnp.dot(x.astype(jnp.bfloat16), w5_ref[...],
                  preferred_element_type=f32) + b5_ref[...]
    gg = jnp.dot(g_ref[...].astype(jnp.bfloat16), hw_ref[...],
                 preferred_element_type=f32) + hb_ref[...]
    gg = jnp.maximum(gg, 0.1 * gg)
    y = (jnp.dot(xf1.astype(jnp.bfloat16), wa_ref[...],
                 preferred_element_type=f32)
         + jnp.dot(gg.astype(jnp.bfloat16), bt_ref[...],
                   preferred_element_type=f32) + b1_ref[...])
    yb = y.astype(jnp.bfloat16)
    out_ref[...] = yb
    _acc_stats(i, yb, stout_ref)

  y9, s9 = pl.pallas_call(
      k9_body,
      grid=(_GRID,),
      in_specs=[_row_spec(512), _full_spec(8, 512), _row_spec(64),
                _full_spec(1, 512), _full_spec(1, 512),
                _full_spec(512, 256), _full_spec(1, 256),
                _full_spec(64, 512), _full_spec(1, 512),
                _full_spec(256, 512), _full_spec(512, 512), _full_spec(1, 512)],
      out_specs=[_row_spec(512), _full_spec(8, 512)],
      out_shape=[jax.ShapeDtypeStruct((_M, 512), jnp.bfloat16),
                 jax.ShapeDtypeStruct((8, 512), f32)],
      compiler_params=pltpu.CompilerParams(dimension_semantics=("arbitrary",)),
  )(y8, s8, g_faces, gam("fe4"), bet("fe4"), wt("fe5"), bias("fe5"),
    hw_blk, hb_blk, wt1a, bt, bias("t1"))

  y10, s10 = _std_layer(y9, s9, gam("t1"), bet("t1"), wt("t2"), bias("t2"),
                        "inorm", "relu", 512, 256)
  y11, s11 = _std_layer(y10, s10, gam("t2"), bet("t2"), wt("t3"), bias("t3"),
                        "inorm", "relu", 256, 128)
  y12, s12 = _std_layer(y11, s11, gam("t3"), bet("t3"), wt("t4"), bias("t4"),
                        "inorm", "relu", 128, 64)

  def k13_body(y_ref, st_ref, g_ref, be_ref, w_ref, b_ref, out_ref):
    x = _norm_act(y_ref[...], st_ref, g_ref[...], be_ref[...],
                  "inorm", "relu", 64)
    z = jnp.dot(x.astype(jnp.bfloat16), w_ref[...],
                preferred_element_type=f32) + b_ref[...]
    zm = jnp.max(z, axis=1, keepdims=True)
    lse = zm + jnp.log(jnp.sum(jnp.exp(z - zm), axis=1, keepdims=True))
    out_ref[...] = z - lse

  out = pl.pallas_call(
      k13_body,
      grid=(_GRID,),
      in_specs=[_row_spec(64), _full_spec(8, 64), _full_spec(1, 64),
                _full_spec(1, 64), _full_spec(64, 2), _full_spec(1, 2)],
      out_specs=_row_spec(2),
      out_shape=jax.ShapeDtypeStruct((_M, 2), f32),
      compiler_params=pltpu.CompilerParams(dimension_semantics=("arbitrary",)),
  )(y12, s12, gam("t4"), bet("t4"), wt("t5"), bias("t5"))

  return out


# speedup vs baseline: 3.1166x; 1.0038x over previous
import functools

import jax
import jax.numpy as jnp
from jax import lax
from jax.experimental import pallas as pl
from jax.experimental.pallas import tpu as pltpu
from jax.experimental.pallas import tpu_sc as plsc

_N = 100000
_M = 100000
_T = 5000
_GRID = _M // _T
_DL = 16
_NW = 32
_CH = 3200
_NCH = 4
_BPW = _CH * _NCH
_BPAD = _NW * _BPW
_MPAD = _BPAD // 4
_EPS = 1e-5


def _sc_gather(table, idx):
  mesh = plsc.VectorSubcoreMesh(core_axis_name="c", subcore_axis_name="s")

  @functools.partial(
      pl.kernel,
      out_type=jax.ShapeDtypeStruct((_BPAD, _DL), jnp.float32),
      mesh=mesh,
      scratch_types=[
          pltpu.VMEM((2, _CH), jnp.int32),
          pltpu.VMEM((2, _CH, _DL), jnp.float32),
          pltpu.SemaphoreType.DMA((2,)),
          pltpu.SemaphoreType.DMA((2,)),
      ],
      compiler_params=pltpu.CompilerParams(use_tc_tiling_on_sc=False),
  )
  def k(table_hbm, idx_hbm, out_hbm, idx_v, rows_v, gsem, ssem):
    wid = lax.axis_index("s") * 2 + lax.axis_index("c")
    base = wid * _BPW

    def start(i):
      b = i % 2
      pltpu.sync_copy(idx_hbm.at[pl.ds(base + i * _CH, _CH)], idx_v.at[b])
      return pltpu.async_copy(table_hbm.at[idx_v.at[b]], rows_v.at[b],
                              gsem.at[b])

    gath = [start(0), None]
    stor = [None, None]
    for i in range(_NCH):
      b, nb = i % 2, (i + 1) % 2
      if i + 1 < _NCH:
        if stor[nb] is not None:
          stor[nb].wait()
        gath[nb] = start(i + 1)
      gath[b].wait()
      stor[b] = pltpu.async_copy(rows_v.at[b],
                                 out_hbm.at[pl.ds(base + i * _CH, _CH)],
                                 ssem.at[b])
    stor[0].wait()
    stor[1].wait()

  return k(table, idx)


def _norm_act(y, st_ref, g, be, kind, act, c_in):
  y = y.astype(jnp.float32)
  if kind == "inorm":
    mu = st_ref[0:1, :] / _M
    var = st_ref[1:2, :] / _M - mu * mu
    x = (y - mu) * lax.rsqrt(var + _EPS) * g + be
  elif kind == "gnorm":
    cnt = float(_M * c_in)
    mu = jnp.sum(st_ref[0:1, :]) / cnt
    var = jnp.sum(st_ref[1:2, :]) / cnt - mu * mu
    x = (y - mu) * lax.rsqrt(var + _EPS) * g + be
  else:
    x = y
  if act == "lrelu":
    x = jnp.maximum(x, 0.01 * x)
  elif act == "relu":
    x = jnp.maximum(x, 0.0)
  return x


def _acc_stats(i, y, stout_ref):
  y = y.astype(jnp.float32)
  s0 = jnp.sum(y, axis=0, keepdims=True)
  s1 = jnp.sum(y * y, axis=0, keepdims=True)
  pad = jnp.zeros((6, y.shape[1]), jnp.float32)
  upd = jnp.concatenate([s0, s1, pad], axis=0)

  @pl.when(i == 0)
  def _():
    stout_ref[...] = jnp.zeros_like(stout_ref)

  stout_ref[...] += upd


def _row_spec(c):
  return pl.BlockSpec((_T, c), lambda i: (i, 0))


def _full_spec(r, c):
  return pl.BlockSpec((r, c), lambda i: (0, 0))


def _std_layer(y_in, stats, g, be, wt, b, kind, act, c_in, c_out):
  def body(y_ref, st_ref, g_ref, be_ref, w_ref, b_ref, out_ref, stout_ref):
    i = pl.program_id(0)
    x = _norm_act(y_ref[...], st_ref, g_ref[...], be_ref[...], kind, act, c_in)
    y = jnp.dot(x, w_ref[...], preferred_element_type=jnp.float32) + b_ref[...]
    yb = y.astype(jnp.bfloat16)
    out_ref[...] = yb
    _acc_stats(i, yb, stout_ref)

  return pl.pallas_call(
      body,
      grid=(_GRID,),
      in_specs=[_row_spec(c_in), _full_spec(8, c_in), _full_spec(1, c_in),
                _full_spec(1, c_in), _full_spec(c_in, c_out), _full_spec(1, c_out)],
      out_specs=[_row_spec(c_out), _full_spec(8, c_out)],
      out_shape=[jax.ShapeDtypeStruct((_M, c_out), jnp.bfloat16),
                 jax.ShapeDtypeStruct((8, c_out), jnp.float32)],
      compiler_params=pltpu.CompilerParams(
          dimension_semantics=("arbitrary",)),
  )(y_in, stats, g, be, wt, b)


def kernel(features, faces, f_infos, params):
  p = params
  f32 = jnp.float32

  def wt(n):
    return p[n + "_W"].T

  def bias(n):
    return p[n + "_b"].reshape(1, -1)

  def gam(n):
    return p[n + "_g"].reshape(1, -1)

  def bet(n):
    return p[n + "_be"].reshape(1, -1)

  w_fe1 = p["fe1_W"]
  a_mat = jnp.zeros((4 * _DL, 128), f32)
  for v in range(4):
    a_mat = a_mat.at[v * _DL:v * _DL + 5, :].set(w_fe1[:, v * 5:(v + 1) * 5].T)
  bw_mat = jnp.zeros((_DL, 128), f32).at[:13, :].set(w_fe1[:, 20:33].T)

  head_w = p["head_W"]
  hw_blk = jnp.zeros((4 * _DL, 512), f32)
  for v in range(4):
    hw_blk = hw_blk.at[v * _DL:v * _DL + 5, v * 128:(v + 1) * 128].set(head_w)
  hb_blk = jnp.tile(p["head_b"], 4).reshape(1, 512)

  w_t1 = p["t1_W"]
  wt1a = w_t1[:, :256].T
  bt = w_t1[:, 256:].reshape(512, 128, 4).transpose(2, 1, 0).reshape(512, 512)

  table = jnp.pad(features, ((0, 0), (0, _DL - features.shape[1])))
  idx = jnp.pad(faces.reshape(-1).astype(jnp.int32), (0, _BPAD - 4 * _M))
  gz = _sc_gather(table, idx)
  g_faces = gz.reshape(_MPAD, 4 * _DL)
  f_pad = jnp.pad(f_infos, ((0, 0), (0, 3)))

  def k1_body(g_ref, f_ref, a_ref, bw_ref, b_ref, out_ref, stout_ref):
    i = pl.program_id(0)
    y = (jnp.dot(g_ref[...], a_ref[...], preferred_element_type=f32)
         + jnp.dot(f_ref[...], bw_ref[...], preferred_element_type=f32)
         + b_ref[...])
    yb = y.astype(jnp.bfloat16)
    out_ref[...] = yb
    _acc_stats(i, yb, stout_ref)

  y1, s1 = pl.pallas_call(
      k1_body,
      grid=(_GRID,),
      in_specs=[_row_spec(64), _row_spec(16), _full_spec(64, 128),
                _full_spec(16, 128), _full_spec(1, 128)],
      out_specs=[_row_spec(128), _full_spec(8, 128)],
      out_shape=[jax.ShapeDtypeStruct((_M, 128), jnp.bfloat16),
                 jax.ShapeDtypeStruct((8, 128), f32)],
      compiler_params=pltpu.CompilerParams(dimension_semantics=("arbitrary",)),
  )(g_faces, f_pad, a_mat, bw_mat, bias("fe1"))

  y2, s2 = _std_layer(y1, s1, gam("fe1"), bet("fe1"), wt("fe2"), bias("fe2"),
                      "inorm", "lrelu", 128, 128)
  y3, s3 = _std_layer(y2, s2, gam("fe2"), bet("fe2"), wt("pr1"), bias("pr1"),
                      "inorm", "lrelu", 128, 256)
  y4, s4 = _std_layer(y3, s3, gam("pr1"), bet("pr1"), wt("pr2"), bias("pr2"),
                      "gnorm", "relu", 256, 512)
  y5, s5 = _std_layer(y4, s4, gam("pr2"), bet("pr2"), wt("pr3"), bias("pr3"),
                      "gnorm", "relu", 512, 512)
  y6, s6 = _std_layer(y5, s5, gam("pr3"), bet("pr3"), wt("pr4"), bias("pr4"),
                      "gnorm", "relu", 512, 256)

  def k7_body(y6_ref, s6_ref, y2_ref, s2_ref, g4_ref, be4_ref, w5_ref, b5_ref,
              g2_ref, be2_ref, w3_ref, b3_ref, out_ref, stout_ref):
    i = pl.program_id(0)
    x = _norm_act(y6_ref[...], s6_ref, g4_ref[...], be4_ref[...],
                  "gnorm", "relu", 256)
    r = jnp.dot(x, w5_ref[...], preferred_element_type=f32) + b5_ref[...]
    h = _norm_act(y2_ref[...], s2_ref, g2_ref[...], be2_ref[...],
                  "inorm", "lrelu", 128) + r
    y = jnp.dot(h, w3_ref[...], preferred_element_type=f32) + b3_ref[...]
    yb = y.astype(jnp.bfloat16)
    out_ref[...] = yb
    _acc_stats(i, yb, stout_ref)

  y7, s7 = pl.pallas_call(
      k7_body,
      grid=(_GRID,),
      in_specs=[_row_spec(256), _full_spec(8, 256), _row_spec(128),
                _full_spec(8, 128), _full_spec(1, 256), _full_spec(1, 256),
                _full_spec(256, 128), _full_spec(1, 128), _full_spec(1, 128),
                _full_spec(1, 128), _full_spec(128, 256), _full_spec(1, 256)],
      out_specs=[_row_spec(256), _full_spec(8, 256)],
      out_shape=[jax.ShapeDtypeStruct((_M, 256), jnp.bfloat16),
                 jax.ShapeDtypeStruct((8, 256), f32)],
      compiler_params=pltpu.CompilerParams(dimension_semantics=("arbitrary",)),
  )(y6, s6, y2, s2, gam("pr4"), bet("pr4"), wt("pr5"), bias("pr5"),
    gam("fe2"), bet("fe2"), wt("fe3"), bias("fe3"))

  y8, s8 = _std_layer(y7, s7, gam("fe3"), bet("fe3"), wt("fe4"), bias("fe4"),
                      "inorm", "lrelu", 256, 512)

  def k9_body(y8_ref, s8_ref, g_ref, g4_ref, be4_ref, w5_ref, b5_ref,
              hw_ref, hb_ref, wa_ref, bt_ref, b1_ref, out_ref, stout_ref):
    i = pl.program_id(0)
    x = _norm_act(y8_ref[...], s8_ref, g4_ref[...], be4_ref[...],
                  "inorm", "lrelu", 512)
    xf1 = jnp.dot(x, w5_ref[...], preferred_element_type=f32) + b5_ref[...]
    gg = jnp.dot(g_ref[...], hw_ref[...], preferred_element_type=f32) + hb_ref[...]
    gg = jnp.maximum(gg, 0.1 * gg)
    y = (jnp.dot(xf1, wa_ref[...], preferred_element_type=f32)
         + jnp.dot(gg, bt_ref[...], preferred_element_type=f32) + b1_ref[...])
    yb = y.astype(jnp.bfloat16)
    out_ref[...] = yb
    _acc_stats(i, yb, stout_ref)

  y9, s9 = pl.pallas_call(
      k9_body,
      grid=(_GRID,),
      in_specs=[_row_spec(512), _full_spec(8, 512), _row_spec(64),
                _full_spec(1, 512), _full_spec(1, 512),
                _full_spec(512, 256), _full_spec(1, 256),
                _full_spec(64, 512), _full_spec(1, 512),
                _full_spec(256, 512), _full_spec(512, 512), _full_spec(1, 512)],
      out_specs=[_row_spec(512), _full_spec(8, 512)],
      out_shape=[jax.ShapeDtypeStruct((_M, 512), jnp.bfloat16),
                 jax.ShapeDtypeStruct((8, 512), f32)],
      compiler_params=pltpu.CompilerParams(dimension_semantics=("arbitrary",)),
  )(y8, s8, g_faces, gam("fe4"), bet("fe4"), wt("fe5"), bias("fe5"),
    hw_blk, hb_blk, wt1a, bt, bias("t1"))

  y10, s10 = _std_layer(y9, s9, gam("t1"), bet("t1"), wt("t2"), bias("t2"),
                        "inorm", "relu", 512, 256)
  y11, s11 = _std_layer(y10, s10, gam("t2"), bet("t2"), wt("t3"), bias("t3"),
                        "inorm", "relu", 256, 128)
  y12, s12 = _std_layer(y11, s11, gam("t3"), bet("t3"), wt("t4"), bias("t4"),
                        "inorm", "relu", 128, 64)

  def k13_body(y_ref, st_ref, g_ref, be_ref, w_ref, b_ref, out_ref):
    x = _norm_act(y_ref[...], st_ref, g_ref[...], be_ref[...],
                  "inorm", "relu", 64)
    z = jnp.dot(x, w_ref[...], preferred_element_type=f32) + b_ref[...]
    zm = jnp.max(z, axis=1, keepdims=True)
    lse = zm + jnp.log(jnp.sum(jnp.exp(z - zm), axis=1, keepdims=True))
    out_ref[...] = z - lse

  out = pl.pallas_call(
      k13_body,
      grid=(_GRID,),
      in_specs=[_row_spec(64), _full_spec(8, 64), _full_spec(1, 64),
                _full_spec(1, 64), _full_spec(64, 2), _full_spec(1, 2)],
      out_specs=_row_spec(2),
      out_shape=jax.ShapeDtypeStruct((_M, 2), f32),
      compiler_params=pltpu.CompilerParams(dimension_semantics=("arbitrary",)),
  )(y12, s12, gam("t4"), bet("t4"), wt("t5"), bias("t5"))

  return out
